# Initial kernel scaffold; baseline (speedup 1.0000x reference)
import jax
import jax.numpy as jnp
from jax.experimental import pallas as pl


def kernel(x_var, x_constr, edge_index_v2c, edge_index_c2v, edge_attr, We_v, be_v, W1_v, b1_v, W2_v, b2_v, We_c, be_c, W1_c, b1_c, W2_c, b2_c, g_var, bt_var, g_constr, bt_constr):
    raise NotImplementedError("write your pallas kernel here")


# speedup vs baseline: 1.0862x; 1.0862x over previous
import functools

import jax
import jax.numpy as jnp
from jax import lax
from jax.experimental import pallas as pl
from jax.experimental.pallas import tpu as pltpu
from jax.experimental.pallas import tpu_sc as plsc

N_NODE = 10000
D = 128
E_TOT = 320000
ED = 4

NC = 1
NS = 16
NW = NC * NS
E_PW = E_TOT // NW
K = 80
SB = 2000
NSB = E_PW // SB
CPS = SB // K
N_PAD = 10240
ROWS_PT = N_PAD // NS
ZROWS = 64

_mesh = plsc.VectorSubcoreMesh(core_axis_name="c", subcore_axis_name="s",
                               num_cores=NC)


@functools.partial(
    pl.kernel,
    out_type=jax.ShapeDtypeStruct((NC, N_PAD, D), jnp.float32),
    mesh=_mesh,
    scratch_types=[
        pltpu.VMEM((SB,), jnp.int32),
        pltpu.VMEM((SB,), jnp.int32),
        pltpu.VMEM((SB * ED,), jnp.float32),
        pltpu.VMEM((K, D), jnp.float32),
        pltpu.VMEM((K,), jnp.int32),
        pltpu.VMEM((ED, D), jnp.float32),
        pltpu.VMEM((1, D), jnp.float32),
        pltpu.VMEM((ZROWS, D), jnp.float32),
        pltpu.VMEM_SHARED((N_PAD, D), jnp.float32),
        pltpu.SemaphoreType.DMA,
    ],
)
def _gine_scatter(x_hbm, src_hbm, dst_hbm, attr_hbm, we_hbm, be_hbm, out_hbm,
                  sidx_v, didx_v, attr_v, rows_v, dk_v, we_v, be_v, zbuf_v,
                  accum, sem):
    cid = lax.axis_index("c")
    sid = lax.axis_index("s")
    wid = cid * NS + sid
    ebase = wid * E_PW

    zero16 = jnp.zeros((16,), jnp.float32)

    def _zrow(i, _):
        for j in range(D // 16):
            zbuf_v[i, pl.ds(j * 16, 16)] = zero16
        return 0
    lax.fori_loop(0, ZROWS, _zrow, 0)
    for q in range(ROWS_PT // ZROWS):
        pltpu.sync_copy(zbuf_v, accum.at[pl.ds(sid * ROWS_PT + q * ZROWS, ZROWS)])

    pltpu.sync_copy(we_hbm, we_v)
    pltpu.sync_copy(be_hbm, be_v)

    plsc.subcore_barrier()

    wvec = [[we_v[k, pl.ds(j * 16, 16)] for j in range(D // 16)]
            for k in range(ED)]
    bvec = [be_v[0, pl.ds(j * 16, 16)] for j in range(D // 16)]

    def _superblock(sb, _):
        sbase = ebase + sb * SB
        pltpu.sync_copy(src_hbm.at[pl.ds(sbase, SB)], sidx_v)
        pltpu.sync_copy(dst_hbm.at[pl.ds(sbase, SB)], didx_v)
        pltpu.sync_copy(attr_hbm.at[pl.ds(sbase * ED, SB * ED)], attr_v)

        def _chunk(it, _):
            base = it * K
            for t in range(K // 16):
                dk_v[pl.ds(t * 16, 16)] = didx_v[pl.ds(base + t * 16, 16)]
            pltpu.async_copy(x_hbm.at[sidx_v.at[pl.ds(base, K)]], rows_v,
                             sem).wait()

            def _quad(q, _):
                av = attr_v[pl.ds(base * ED + q * 16, 16)]
                for i in range(4):
                    e = q * 4 + i
                    for j in range(D // 16):
                        sl = pl.ds(j * 16, 16)
                        acc = rows_v[e, sl] + bvec[j]
                        for k in range(ED):
                            acc = acc + av[4 * i + k] * wvec[k][j]
                        rows_v[e, sl] = jnp.maximum(acc, 0.0)
                return 0
            lax.fori_loop(0, K // 4, _quad, 0)

            pltpu.sync_copy(rows_v, accum.at[dk_v], add=True)
            return 0
        lax.fori_loop(0, CPS, _chunk, 0)
        return 0
    lax.fori_loop(0, NSB, _superblock, 0)

    plsc.subcore_barrier()

    for q in range(ROWS_PT // ZROWS):
        r0 = sid * ROWS_PT + q * ZROWS
        pltpu.sync_copy(accum.at[pl.ds(r0, ZROWS)],
                        out_hbm.at[cid, pl.ds(r0, ZROWS)])


BLK = 1000


def _mlp_ln_body(x_ref, a_ref, w1_ref, b1_ref, w2_ref, b2_ref, g_ref, bt_ref,
                 o_ref):
    x = x_ref[...]
    h = x
    for c in range(NC):
        h = h + a_ref[c]
    t = jnp.dot(h, w1_ref[...], preferred_element_type=jnp.float32) + b1_ref[...]
    t = jnp.dot(jnp.maximum(t, 0.0), w2_ref[...],
                preferred_element_type=jnp.float32) + b2_ref[...]
    r = x + t
    mu = jnp.mean(r, axis=1, keepdims=True)
    var = jnp.mean((r - mu) ** 2, axis=1, keepdims=True)
    o_ref[...] = (r - mu) * lax.rsqrt(var + 1e-5) * g_ref[...] + bt_ref[...]


_mlp_ln = pl.pallas_call(
    _mlp_ln_body,
    grid=(N_NODE // BLK,),
    in_specs=[
        pl.BlockSpec((BLK, D), lambda i: (i, 0)),
        pl.BlockSpec((NC, BLK, D), lambda i: (0, i, 0)),
        pl.BlockSpec((D, D), lambda i: (0, 0)),
        pl.BlockSpec((1, D), lambda i: (0, 0)),
        pl.BlockSpec((D, D), lambda i: (0, 0)),
        pl.BlockSpec((1, D), lambda i: (0, 0)),
        pl.BlockSpec((1, D), lambda i: (0, 0)),
        pl.BlockSpec((1, D), lambda i: (0, 0)),
    ],
    out_specs=pl.BlockSpec((BLK, D), lambda i: (i, 0)),
    out_shape=jax.ShapeDtypeStruct((N_NODE, D), jnp.float32),
)


def kernel(x_var, x_constr, edge_index_v2c, edge_index_c2v, edge_attr,
           We_v, be_v, W1_v, b1_v, W2_v, b2_v,
           We_c, be_c, W1_c, b1_c, W2_c, b2_c,
           g_var, bt_var, g_constr, bt_constr):
    s_v2c = edge_index_v2c[0].astype(jnp.int32)
    d_v2c = edge_index_v2c[1].astype(jnp.int32)
    s_c2v = edge_index_c2v[0].astype(jnp.int32)
    d_c2v = edge_index_c2v[1].astype(jnp.int32)
    attr = edge_attr.astype(jnp.float32).reshape(-1)

    r1 = lambda v: v.reshape(1, D)

    agg_c = _gine_scatter(x_var, s_v2c, d_v2c, attr, We_v, r1(be_v))
    xc = _mlp_ln(x_constr, agg_c, W1_v, r1(b1_v), W2_v, r1(b2_v),
                 r1(g_constr), r1(bt_constr))
    agg_v = _gine_scatter(xc, s_c2v, d_c2v, attr, We_c, r1(be_c))
    xv = _mlp_ln(x_var, agg_v, W1_c, r1(b1_c), W2_c, r1(b2_c),
                 r1(g_var), r1(bt_var))
    return (xv, xc)


# speedup vs baseline: 1.2646x; 1.1643x over previous
import functools

import jax
import jax.numpy as jnp
from jax import lax
from jax.experimental import pallas as pl
from jax.experimental.pallas import tpu as pltpu
from jax.experimental.pallas import tpu_sc as plsc

N_NODE = 10000
D = 128
E_TOT = 320000
ED = 4

NC = 1
NS = 16
NW = NC * NS
E_PW = E_TOT // NW
K = 40
SB = 2000
NSB = E_PW // SB
CPS = SB // K
NCH = E_PW // K
NP = NCH // 2
N_PAD = 10240
ROWS_PT = N_PAD // NS
ZROWS = 64
WROWS = 128

_mesh = plsc.VectorSubcoreMesh(core_axis_name="c", subcore_axis_name="s",
                               num_cores=NC)


@functools.partial(
    pl.kernel,
    out_type=jax.ShapeDtypeStruct((NC, N_PAD, D), jnp.float32),
    mesh=_mesh,
    scratch_types=[
        pltpu.VMEM((SB,), jnp.int32),
        pltpu.VMEM((SB,), jnp.int32),
        pltpu.VMEM((SB * ED,), jnp.float32),
        pltpu.VMEM((K, D), jnp.float32),
        pltpu.VMEM((K, D), jnp.float32),
        pltpu.VMEM((K,), jnp.int32),
        pltpu.VMEM((K,), jnp.int32),
        pltpu.VMEM((K,), jnp.int32),
        pltpu.VMEM((K,), jnp.int32),
        pltpu.VMEM((K * ED,), jnp.float32),
        pltpu.VMEM((K * ED,), jnp.float32),
        pltpu.VMEM((ED, D), jnp.float32),
        pltpu.VMEM((1, D), jnp.float32),
        pltpu.VMEM((ZROWS, D), jnp.float32),
        pltpu.VMEM_SHARED((N_PAD, D), jnp.float32),
        pltpu.SemaphoreType.DMA,
        pltpu.SemaphoreType.DMA,
        pltpu.SemaphoreType.DMA,
        pltpu.SemaphoreType.DMA,
    ],
)
def _gine_scatter(x_hbm, src_hbm, dst_hbm, attr_hbm, we_hbm, be_hbm, out_hbm,
                  sidx_v, didx_v, attr_v, rows_a, rows_b, sk_a, sk_b, dk_a,
                  dk_b, ak_a, ak_b, we_v, be_v, zbuf_v, accum,
                  gsem_a, gsem_b, ssem_a, ssem_b):
    cid = lax.axis_index("c")
    sid = lax.axis_index("s")
    wid = cid * NS + sid
    ebase = wid * E_PW

    zero16 = jnp.zeros((16,), jnp.float32)

    def _zrow(i, _):
        for j in range(D // 16):
            zbuf_v[i, pl.ds(j * 16, 16)] = zero16
        return 0
    lax.fori_loop(0, ZROWS, _zrow, 0)
    for q in range(ROWS_PT // ZROWS):
        pltpu.sync_copy(zbuf_v, accum.at[pl.ds(sid * ROWS_PT + q * ZROWS, ZROWS)])

    pltpu.sync_copy(we_hbm, we_v)
    pltpu.sync_copy(be_hbm, be_v)

    plsc.subcore_barrier()

    wvec = [[we_v[k, pl.ds(j * 16, 16)] for j in range(D // 16)]
            for k in range(ED)]
    bvec = [be_v[0, pl.ds(j * 16, 16)] for j in range(D // 16)]

    def _stage_sb(c):
        sbase = ebase + (c // CPS) * SB
        pltpu.sync_copy(src_hbm.at[pl.ds(sbase, SB)], sidx_v)
        pltpu.sync_copy(dst_hbm.at[pl.ds(sbase, SB)], didx_v)
        pltpu.sync_copy(attr_hbm.at[pl.ds(sbase * ED, SB * ED)], attr_v)

    def _prep(c, sk, dk, ak):
        base = (c % CPS) * K
        for t in range((K + 15) // 16):
            o = min(t * 16, K - 16)
            sk[pl.ds(o, 16)] = sidx_v[pl.ds(base + o, 16)]
            dk[pl.ds(o, 16)] = didx_v[pl.ds(base + o, 16)]
        for t in range(K * ED // 16):
            ak[pl.ds(t * 16, 16)] = attr_v[pl.ds(base * ED + t * 16, 16)]

    def _compute(rows, ak):
        def _e8(e8, _):
            for i in range(8):
                e = e8 * 8 + i
                if i % 4 == 0:
                    av = ak[pl.ds(e8 * 32 + (i // 4) * 16, 16)]
                for j in range(D // 16):
                    sl = pl.ds(j * 16, 16)
                    acc = rows[e, sl] + bvec[j]
                    for k in range(ED):
                        acc = acc + av[4 * (i % 4) + k] * wvec[k][j]
                    rows[e, sl] = jnp.maximum(acc, 0.0)
            return 0
        lax.fori_loop(0, K // 8, _e8, 0)

    _stage_sb(0)
    _prep(0, sk_a, dk_a, ak_a)
    ga0 = pltpu.async_copy(x_hbm.at[sk_a], rows_a, gsem_a)

    def _pair(p, _):
        c0 = 2 * p
        c1 = c0 + 1
        c2 = c0 + 2

        @pl.when(p > 0)
        def _():
            pltpu.make_async_copy(rows_b, accum.at[dk_b], ssem_b).wait()
        _prep(c1, sk_b, dk_b, ak_b)
        pltpu.async_copy(x_hbm.at[sk_b], rows_b, gsem_b)

        pltpu.make_async_copy(x_hbm.at[sk_a], rows_a, gsem_a).wait()
        _compute(rows_a, ak_a)
        pltpu.async_copy(rows_a, accum.at[dk_a], ssem_a, add=True)
        pltpu.make_async_copy(rows_a, accum.at[dk_a], ssem_a).wait()

        @pl.when(p < NP - 1)
        def _():
            @pl.when(c2 % CPS == 0)
            def _():
                _stage_sb(c2)
            _prep(c2, sk_a, dk_a, ak_a)
            pltpu.async_copy(x_hbm.at[sk_a], rows_a, gsem_a)

        pltpu.make_async_copy(x_hbm.at[sk_b], rows_b, gsem_b).wait()
        _compute(rows_b, ak_b)
        pltpu.async_copy(rows_b, accum.at[dk_b], ssem_b, add=True)
        return 0
    lax.fori_loop(0, NP, _pair, 0)
    pltpu.make_async_copy(rows_b, accum.at[dk_b], ssem_b).wait()

    plsc.subcore_barrier()

    for q in range(ROWS_PT // WROWS):
        r0 = sid * ROWS_PT + q * WROWS
        pltpu.sync_copy(accum.at[pl.ds(r0, WROWS)],
                        out_hbm.at[cid, pl.ds(r0, WROWS)])


BLK = 1000


def _mlp_ln_body(x_ref, a_ref, w1_ref, b1_ref, w2_ref, b2_ref, g_ref, bt_ref,
                 o_ref):
    x = x_ref[...]
    h = x
    for c in range(NC):
        h = h + a_ref[c]
    t = jnp.dot(h, w1_ref[...], preferred_element_type=jnp.float32) + b1_ref[...]
    t = jnp.dot(jnp.maximum(t, 0.0), w2_ref[...],
                preferred_element_type=jnp.float32) + b2_ref[...]
    r = x + t
    mu = jnp.mean(r, axis=1, keepdims=True)
    var = jnp.mean((r - mu) ** 2, axis=1, keepdims=True)
    o_ref[...] = (r - mu) * lax.rsqrt(var + 1e-5) * g_ref[...] + bt_ref[...]


_mlp_ln = pl.pallas_call(
    _mlp_ln_body,
    grid=(N_NODE // BLK,),
    in_specs=[
        pl.BlockSpec((BLK, D), lambda i: (i, 0)),
        pl.BlockSpec((NC, BLK, D), lambda i: (0, i, 0)),
        pl.BlockSpec((D, D), lambda i: (0, 0)),
        pl.BlockSpec((1, D), lambda i: (0, 0)),
        pl.BlockSpec((D, D), lambda i: (0, 0)),
        pl.BlockSpec((1, D), lambda i: (0, 0)),
        pl.BlockSpec((1, D), lambda i: (0, 0)),
        pl.BlockSpec((1, D), lambda i: (0, 0)),
    ],
    out_specs=pl.BlockSpec((BLK, D), lambda i: (i, 0)),
    out_shape=jax.ShapeDtypeStruct((N_NODE, D), jnp.float32),
)


def kernel(x_var, x_constr, edge_index_v2c, edge_index_c2v, edge_attr,
           We_v, be_v, W1_v, b1_v, W2_v, b2_v,
           We_c, be_c, W1_c, b1_c, W2_c, b2_c,
           g_var, bt_var, g_constr, bt_constr):
    s_v2c = edge_index_v2c[0].astype(jnp.int32)
    d_v2c = edge_index_v2c[1].astype(jnp.int32)
    s_c2v = edge_index_c2v[0].astype(jnp.int32)
    d_c2v = edge_index_c2v[1].astype(jnp.int32)
    attr = edge_attr.astype(jnp.float32).reshape(-1)

    r1 = lambda v: v.reshape(1, D)

    agg_c = _gine_scatter(x_var, s_v2c, d_v2c, attr, We_v, r1(be_v))
    xc = _mlp_ln(x_constr, agg_c, W1_v, r1(b1_v), W2_v, r1(b2_v),
                 r1(g_constr), r1(bt_constr))
    agg_v = _gine_scatter(xc, s_c2v, d_c2v, attr, We_c, r1(be_c))
    xv = _mlp_ln(x_var, agg_v, W1_c, r1(b1_c), W2_c, r1(b2_c),
                 r1(g_var), r1(bt_var))
    return (xv, xc)


# speedup vs baseline: 2.9520x; 2.3343x over previous
import functools

import jax
import jax.numpy as jnp
from jax import lax
from jax.experimental import pallas as pl
from jax.experimental.pallas import tpu as pltpu
from jax.experimental.pallas import tpu_sc as plsc

N_NODE = 10000
D = 128
E_TOT = 320000
ED = 4

NC = 1
NS = 16
NW = NC * NS
E_PW = E_TOT // NW
K = 40
SB = 2000
NSB = E_PW // SB
CPS = SB // K
NCH = E_PW // K
NP = NCH // 2
N_PAD = 10240
ROWS_PT = N_PAD // NS
ZROWS = 64
WROWS = 128

_mesh = plsc.VectorSubcoreMesh(core_axis_name="c", subcore_axis_name="s",
                               num_cores=NC)


@functools.partial(
    pl.kernel,
    out_type=jax.ShapeDtypeStruct((NC, N_PAD, D), jnp.float32),
    mesh=_mesh,
    scratch_types=[
        pltpu.VMEM((SB,), jnp.int32),
        pltpu.VMEM((SB,), jnp.int32),
        pltpu.VMEM((K, D), jnp.float32),
        pltpu.VMEM((K, D), jnp.float32),
        pltpu.VMEM((K, D), jnp.float32),
        pltpu.VMEM((K, D), jnp.float32),
        pltpu.VMEM((K,), jnp.int32),
        pltpu.VMEM((K,), jnp.int32),
        pltpu.VMEM((K,), jnp.int32),
        pltpu.VMEM((K,), jnp.int32),
        pltpu.VMEM((ZROWS, D), jnp.float32),
        pltpu.VMEM_SHARED((N_PAD, D), jnp.float32),
        pltpu.SemaphoreType.DMA,
        pltpu.SemaphoreType.DMA,
        pltpu.SemaphoreType.DMA,
        pltpu.SemaphoreType.DMA,
        pltpu.SemaphoreType.DMA,
        pltpu.SemaphoreType.DMA,
    ],
)
def _gine_scatter(x_hbm, src_hbm, dst_hbm, e_hbm, out_hbm,
                  sidx_v, didx_v, rows_a, rows_b, xr_a, xr_b, sk_a, sk_b,
                  dk_a, dk_b, zbuf_v, accum,
                  gsem_a, gsem_b, esem_a, esem_b, ssem_a, ssem_b):
    cid = lax.axis_index("c")
    sid = lax.axis_index("s")
    wid = cid * NS + sid
    ebase = wid * E_PW

    zero16 = jnp.zeros((16,), jnp.float32)

    def _zrow(i, _):
        for j in range(D // 16):
            zbuf_v[i, pl.ds(j * 16, 16)] = zero16
        return 0
    lax.fori_loop(0, ZROWS, _zrow, 0)
    for q in range(ROWS_PT // ZROWS):
        pltpu.sync_copy(zbuf_v, accum.at[pl.ds(sid * ROWS_PT + q * ZROWS, ZROWS)])

    plsc.subcore_barrier()

    def _stage_sb(c):
        sbase = ebase + (c // CPS) * SB
        pltpu.sync_copy(src_hbm.at[pl.ds(sbase, SB)], sidx_v)
        pltpu.sync_copy(dst_hbm.at[pl.ds(sbase, SB)], didx_v)

    def _launch(c, sk, dk, rows, xr, gsem, esem):
        base = (c % CPS) * K
        for t in range((K + 15) // 16):
            o = min(t * 16, K - 16)
            sk[pl.ds(o, 16)] = sidx_v[pl.ds(base + o, 16)]
            dk[pl.ds(o, 16)] = didx_v[pl.ds(base + o, 16)]
        pltpu.async_copy(e_hbm.at[pl.ds(ebase + c * K, K)], rows, esem)
        pltpu.async_copy(x_hbm.at[sk], xr, gsem)

    def _compute(rows, xr):
        def _e8(e8, _):
            for i in range(8):
                e = e8 * 8 + i
                for j in range(D // 16):
                    sl = pl.ds(j * 16, 16)
                    rows[e, sl] = jnp.maximum(rows[e, sl] + xr[e, sl], 0.0)
            return 0
        lax.fori_loop(0, K // 8, _e8, 0)

    _stage_sb(0)
    _launch(0, sk_a, dk_a, rows_a, xr_a, gsem_a, esem_a)

    def _pair(p, _):
        c0 = 2 * p
        c1 = c0 + 1
        c2 = c0 + 2

        @pl.when(p > 0)
        def _():
            pltpu.make_async_copy(rows_b, accum.at[dk_b], ssem_b).wait()
        _launch(c1, sk_b, dk_b, rows_b, xr_b, gsem_b, esem_b)

        pltpu.make_async_copy(e_hbm.at[pl.ds(ebase + c0 * K, K)], rows_a,
                              esem_a).wait()
        pltpu.make_async_copy(x_hbm.at[sk_a], xr_a, gsem_a).wait()
        _compute(rows_a, xr_a)
        pltpu.async_copy(rows_a, accum.at[dk_a], ssem_a, add=True)
        pltpu.make_async_copy(rows_a, accum.at[dk_a], ssem_a).wait()

        @pl.when(p < NP - 1)
        def _():
            @pl.when(c2 % CPS == 0)
            def _():
                _stage_sb(c2)
            _launch(c2, sk_a, dk_a, rows_a, xr_a, gsem_a, esem_a)

        pltpu.make_async_copy(e_hbm.at[pl.ds(ebase + c1 * K, K)], rows_b,
                              esem_b).wait()
        pltpu.make_async_copy(x_hbm.at[sk_b], xr_b, gsem_b).wait()
        _compute(rows_b, xr_b)
        pltpu.async_copy(rows_b, accum.at[dk_b], ssem_b, add=True)
        return 0
    lax.fori_loop(0, NP, _pair, 0)
    pltpu.make_async_copy(rows_b, accum.at[dk_b], ssem_b).wait()

    plsc.subcore_barrier()

    for q in range(ROWS_PT // WROWS):
        r0 = sid * ROWS_PT + q * WROWS
        pltpu.sync_copy(accum.at[pl.ds(r0, WROWS)],
                        out_hbm.at[cid, pl.ds(r0, WROWS)])


BE = 8000
BLK = 1000


def _eproj_body(at_ref, wev_ref, bev_ref, wec_ref, bec_ref, ov_ref, oc_ref):
    a = at_ref[...]
    dn = (((1,), (0,)), ((), ()))
    ov_ref[...] = lax.dot_general(a, wev_ref[...], dn,
                                  preferred_element_type=jnp.float32) + bev_ref[...]
    oc_ref[...] = lax.dot_general(a, wec_ref[...], dn,
                                  preferred_element_type=jnp.float32) + bec_ref[...]


_eproj = pl.pallas_call(
    _eproj_body,
    grid=(E_TOT // BE,),
    in_specs=[
        pl.BlockSpec((BE, ED), lambda i: (i, 0)),
        pl.BlockSpec((ED, D), lambda i: (0, 0)),
        pl.BlockSpec((1, D), lambda i: (0, 0)),
        pl.BlockSpec((ED, D), lambda i: (0, 0)),
        pl.BlockSpec((1, D), lambda i: (0, 0)),
    ],
    out_specs=[
        pl.BlockSpec((BE, D), lambda i: (i, 0)),
        pl.BlockSpec((BE, D), lambda i: (i, 0)),
    ],
    out_shape=[
        jax.ShapeDtypeStruct((E_TOT, D), jnp.float32),
        jax.ShapeDtypeStruct((E_TOT, D), jnp.float32),
    ],
)


def _mlp_ln_body(x_ref, a_ref, w1_ref, b1_ref, w2_ref, b2_ref, g_ref, bt_ref,
                 o_ref):
    x = x_ref[...]
    h = x
    for c in range(NC):
        h = h + a_ref[c]
    t = jnp.dot(h, w1_ref[...], preferred_element_type=jnp.float32) + b1_ref[...]
    t = jnp.dot(jnp.maximum(t, 0.0), w2_ref[...],
                preferred_element_type=jnp.float32) + b2_ref[...]
    r = x + t
    mu = jnp.mean(r, axis=1, keepdims=True)
    var = jnp.mean((r - mu) ** 2, axis=1, keepdims=True)
    o_ref[...] = (r - mu) * lax.rsqrt(var + 1e-5) * g_ref[...] + bt_ref[...]


_mlp_ln = pl.pallas_call(
    _mlp_ln_body,
    grid=(N_NODE // BLK,),
    in_specs=[
        pl.BlockSpec((BLK, D), lambda i: (i, 0)),
        pl.BlockSpec((NC, BLK, D), lambda i: (0, i, 0)),
        pl.BlockSpec((D, D), lambda i: (0, 0)),
        pl.BlockSpec((1, D), lambda i: (0, 0)),
        pl.BlockSpec((D, D), lambda i: (0, 0)),
        pl.BlockSpec((1, D), lambda i: (0, 0)),
        pl.BlockSpec((1, D), lambda i: (0, 0)),
        pl.BlockSpec((1, D), lambda i: (0, 0)),
    ],
    out_specs=pl.BlockSpec((BLK, D), lambda i: (i, 0)),
    out_shape=jax.ShapeDtypeStruct((N_NODE, D), jnp.float32),
)


def kernel(x_var, x_constr, edge_index_v2c, edge_index_c2v, edge_attr,
           We_v, be_v, W1_v, b1_v, W2_v, b2_v,
           We_c, be_c, W1_c, b1_c, W2_c, b2_c,
           g_var, bt_var, g_constr, bt_constr):
    s_v2c = edge_index_v2c[0].astype(jnp.int32)
    d_v2c = edge_index_v2c[1].astype(jnp.int32)
    s_c2v = edge_index_c2v[0].astype(jnp.int32)
    d_c2v = edge_index_c2v[1].astype(jnp.int32)

    r1 = lambda v: v.reshape(1, D)

    e_v, e_c = _eproj(edge_attr.astype(jnp.float32), We_v, r1(be_v),
                      We_c, r1(be_c))

    agg_c = _gine_scatter(x_var, s_v2c, d_v2c, e_v)
    xc = _mlp_ln(x_constr, agg_c, W1_v, r1(b1_v), W2_v, r1(b2_v),
                 r1(g_constr), r1(bt_constr))
    agg_v = _gine_scatter(xc, s_c2v, d_c2v, e_c)
    xv = _mlp_ln(x_var, agg_v, W1_c, r1(b1_c), W2_c, r1(b2_c),
                 r1(g_var), r1(bt_var))
    return (xv, xc)


# speedup vs baseline: 3.2283x; 1.0936x over previous
import functools

import jax
import jax.numpy as jnp
from jax import lax
from jax.experimental import pallas as pl
from jax.experimental.pallas import tpu as pltpu
from jax.experimental.pallas import tpu_sc as plsc

N_NODE = 10000
D = 128
E_TOT = 320000
ED = 4

NC = 1
NS = 16
NW = NC * NS
E_PW = E_TOT // NW
K = 40
SB = 2000
NSB = E_PW // SB
CPS = SB // K
NCH = E_PW // K
NTRI = (NCH - 2) // 3
NB = 3
N_PAD = 10240
ROWS_PT = N_PAD // NS
ZROWS = 64
WROWS = 128

_mesh = plsc.VectorSubcoreMesh(core_axis_name="c", subcore_axis_name="s",
                               num_cores=NC)


@functools.partial(
    pl.kernel,
    out_type=jax.ShapeDtypeStruct((NC, N_PAD, D), jnp.float32),
    mesh=_mesh,
    scratch_types=[
        pltpu.VMEM((SB,), jnp.int32),
        pltpu.VMEM((SB,), jnp.int32),
        [pltpu.VMEM((K, D), jnp.float32) for _ in range(NB)],
        [pltpu.VMEM((K, D), jnp.float32) for _ in range(NB)],
        [pltpu.VMEM((K,), jnp.int32) for _ in range(NB)],
        [pltpu.VMEM((K,), jnp.int32) for _ in range(NB)],
        pltpu.VMEM((ZROWS, D), jnp.float32),
        pltpu.VMEM_SHARED((N_PAD, D), jnp.float32),
        [pltpu.SemaphoreType.DMA for _ in range(NB)],
        [pltpu.SemaphoreType.DMA for _ in range(NB)],
        [pltpu.SemaphoreType.DMA for _ in range(NB)],
    ],
)
def _gine_scatter(x_hbm, src_hbm, dst_hbm, e_hbm, out_hbm,
                  sidx_v, didx_v, rows, xr, sk, dk, zbuf_v, accum,
                  gsem, esem, ssem):
    cid = lax.axis_index("c")
    sid = lax.axis_index("s")
    wid = cid * NS + sid
    ebase = wid * E_PW

    zero16 = jnp.zeros((16,), jnp.float32)

    def _zrow(i, _):
        for j in range(D // 16):
            zbuf_v[i, pl.ds(j * 16, 16)] = zero16
        return 0
    lax.fori_loop(0, ZROWS, _zrow, 0)
    for q in range(ROWS_PT // ZROWS):
        pltpu.sync_copy(zbuf_v, accum.at[pl.ds(sid * ROWS_PT + q * ZROWS, ZROWS)])

    plsc.subcore_barrier()

    def _stage_sb(c):
        sbase = ebase + (c // CPS) * SB
        pltpu.sync_copy(src_hbm.at[pl.ds(sbase, SB)], sidx_v)
        pltpu.sync_copy(dst_hbm.at[pl.ds(sbase, SB)], didx_v)

    def _launch(c, r):
        base = (c % CPS) * K
        for t in range((K + 15) // 16):
            o = min(t * 16, K - 16)
            sk[r][pl.ds(o, 16)] = sidx_v[pl.ds(base + o, 16)]
            dk[r][pl.ds(o, 16)] = didx_v[pl.ds(base + o, 16)]
        pltpu.async_copy(e_hbm.at[pl.ds(ebase + c * K, K)], rows[r], esem[r])
        pltpu.async_copy(x_hbm.at[sk[r]], xr[r], gsem[r])

    def _compute(r):
        def _e8(e8, _):
            for i in range(8):
                e = e8 * 8 + i
                for j in range(D // 16):
                    sl = pl.ds(j * 16, 16)
                    rows[r][e, sl] = jnp.maximum(
                        rows[r][e, sl] + xr[r][e, sl], 0.0)
            return 0
        lax.fori_loop(0, K // 8, _e8, 0)

    def _process(c, r, launch_ahead):
        pltpu.make_async_copy(e_hbm.at[pl.ds(ebase + c * K, K)], rows[r],
                              esem[r]).wait()
        pltpu.make_async_copy(x_hbm.at[sk[r]], xr[r], gsem[r]).wait()
        _compute(r)
        pltpu.async_copy(rows[r], accum.at[dk[r]], ssem[r], add=True)
        if launch_ahead:
            r2 = (r + 2) % NB
            @pl.when(c >= 1)
            def _():
                pltpu.make_async_copy(rows[r2], accum.at[dk[r2]],
                                      ssem[r2]).wait()
            c2 = c + 2
            @pl.when(c2 % CPS == 0)
            def _():
                _stage_sb(c2)
            _launch(c2, r2)

    _stage_sb(0)
    _launch(0, 0)
    _launch(1, 1)

    def _triple(t, _):
        for r in range(NB):
            _process(3 * t + r, r, True)
        return 0
    lax.fori_loop(0, NTRI, _triple, 0)
    _process(NCH - 2, (NCH - 2) % NB, False)
    _process(NCH - 1, (NCH - 1) % NB, False)
    for r in range(NB):
        pltpu.make_async_copy(rows[r], accum.at[dk[r]], ssem[r]).wait()

    plsc.subcore_barrier()

    for q in range(ROWS_PT // WROWS):
        r0 = sid * ROWS_PT + q * WROWS
        pltpu.sync_copy(accum.at[pl.ds(r0, WROWS)],
                        out_hbm.at[cid, pl.ds(r0, WROWS)])


BE = 8000
BLK = 1000


def _eproj_body(at_ref, wev_ref, bev_ref, wec_ref, bec_ref, ov_ref, oc_ref):
    a = at_ref[...]
    dn = (((1,), (0,)), ((), ()))
    ov_ref[...] = lax.dot_general(a, wev_ref[...], dn,
                                  preferred_element_type=jnp.float32) + bev_ref[...]
    oc_ref[...] = lax.dot_general(a, wec_ref[...], dn,
                                  preferred_element_type=jnp.float32) + bec_ref[...]


_eproj = pl.pallas_call(
    _eproj_body,
    grid=(E_TOT // BE,),
    in_specs=[
        pl.BlockSpec((BE, ED), lambda i: (i, 0)),
        pl.BlockSpec((ED, D), lambda i: (0, 0)),
        pl.BlockSpec((1, D), lambda i: (0, 0)),
        pl.BlockSpec((ED, D), lambda i: (0, 0)),
        pl.BlockSpec((1, D), lambda i: (0, 0)),
    ],
    out_specs=[
        pl.BlockSpec((BE, D), lambda i: (i, 0)),
        pl.BlockSpec((BE, D), lambda i: (i, 0)),
    ],
    out_shape=[
        jax.ShapeDtypeStruct((E_TOT, D), jnp.float32),
        jax.ShapeDtypeStruct((E_TOT, D), jnp.float32),
    ],
)


def _mlp_ln_body(x_ref, a_ref, w1_ref, b1_ref, w2_ref, b2_ref, g_ref, bt_ref,
                 o_ref):
    x = x_ref[...]
    h = x
    for c in range(NC):
        h = h + a_ref[c]
    t = jnp.dot(h, w1_ref[...], preferred_element_type=jnp.float32) + b1_ref[...]
    t = jnp.dot(jnp.maximum(t, 0.0), w2_ref[...],
                preferred_element_type=jnp.float32) + b2_ref[...]
    r = x + t
    mu = jnp.mean(r, axis=1, keepdims=True)
    var = jnp.mean((r - mu) ** 2, axis=1, keepdims=True)
    o_ref[...] = (r - mu) * lax.rsqrt(var + 1e-5) * g_ref[...] + bt_ref[...]


_mlp_ln = pl.pallas_call(
    _mlp_ln_body,
    grid=(N_NODE // BLK,),
    in_specs=[
        pl.BlockSpec((BLK, D), lambda i: (i, 0)),
        pl.BlockSpec((NC, BLK, D), lambda i: (0, i, 0)),
        pl.BlockSpec((D, D), lambda i: (0, 0)),
        pl.BlockSpec((1, D), lambda i: (0, 0)),
        pl.BlockSpec((D, D), lambda i: (0, 0)),
        pl.BlockSpec((1, D), lambda i: (0, 0)),
        pl.BlockSpec((1, D), lambda i: (0, 0)),
        pl.BlockSpec((1, D), lambda i: (0, 0)),
    ],
    out_specs=pl.BlockSpec((BLK, D), lambda i: (i, 0)),
    out_shape=jax.ShapeDtypeStruct((N_NODE, D), jnp.float32),
)


def kernel(x_var, x_constr, edge_index_v2c, edge_index_c2v, edge_attr,
           We_v, be_v, W1_v, b1_v, W2_v, b2_v,
           We_c, be_c, W1_c, b1_c, W2_c, b2_c,
           g_var, bt_var, g_constr, bt_constr):
    s_v2c = edge_index_v2c[0].astype(jnp.int32)
    d_v2c = edge_index_v2c[1].astype(jnp.int32)
    s_c2v = edge_index_c2v[0].astype(jnp.int32)
    d_c2v = edge_index_c2v[1].astype(jnp.int32)

    r1 = lambda v: v.reshape(1, D)

    e_v, e_c = _eproj(edge_attr.astype(jnp.float32), We_v, r1(be_v),
                      We_c, r1(be_c))

    agg_c = _gine_scatter(x_var, s_v2c, d_v2c, e_v)
    xc = _mlp_ln(x_constr, agg_c, W1_v, r1(b1_v), W2_v, r1(b2_v),
                 r1(g_constr), r1(bt_constr))
    agg_v = _gine_scatter(xc, s_c2v, d_c2v, e_c)
    xv = _mlp_ln(x_var, agg_v, W1_c, r1(b1_c), W2_c, r1(b2_c),
                 r1(g_var), r1(bt_var))
    return (xv, xc)
